# baseline (speedup 1.0000x reference)
import functools

import jax
import jax.numpy as jnp
from jax import lax
from jax.experimental import pallas as pl
from jax.experimental.pallas import tpu as pltpu
from jax.experimental.pallas import tpu_sc as plsc

_NUM_TOKENS = 32768
_DIM = 2048
_NUM_EXPERTS = 8
_TOP_K = 2
_BLK = 2048


def _router_body(x_ref, wt_ref, bias_ref, out_ref, cnt_ref):
    i = pl.program_id(0)
    x = x_ref[...]
    wt = wt_ref[...]
    logits = jax.lax.dot_general(
        x, wt, (((1,), (0,)), ((), ())),
        preferred_element_type=jnp.float32,
    )
    lt = logits.T
    scores = jax.nn.sigmoid(lt)
    biased = scores + bias_ref[...]

    row = jax.lax.broadcasted_iota(jnp.int32, biased.shape, 0)
    m1 = jnp.max(biased, axis=0, keepdims=True)
    i1 = jnp.min(jnp.where(biased == m1, row, _NUM_EXPERTS),
                 axis=0, keepdims=True)
    masked = jnp.where(row == i1, -jnp.inf, biased)
    m2 = jnp.max(masked, axis=0, keepdims=True)
    i2 = jnp.min(jnp.where(masked == m2, row, _NUM_EXPERTS),
                 axis=0, keepdims=True)

    sel1 = row == i1
    sel2 = row == i2
    raw1 = jnp.sum(jnp.where(sel1, scores, 0.0), axis=0, keepdims=True)
    raw2 = jnp.sum(jnp.where(sel2, scores, 0.0), axis=0, keepdims=True)
    denom = raw1 + raw2 + 1e-20
    out_ref[...] = jnp.concatenate(
        [raw1 / denom, raw2 / denom,
         jax.lax.bitcast_convert_type(i1, jnp.float32),
         jax.lax.bitcast_convert_type(i2, jnp.float32)], axis=0)

    counts = jnp.sum(
        jnp.where(sel1, 1.0, 0.0) + jnp.where(sel2, 1.0, 0.0),
        axis=1, keepdims=True)

    @pl.when(i == 0)
    def _init():
        cnt_ref[...] = counts

    @pl.when(i != 0)
    def _accum():
        cnt_ref[...] += counts


_NW = 32
_RPW = 8192 // _NW
_CH = 16
_NCH = _RPW // _CH


def _sc_stream_probe(x):
    mesh = plsc.VectorSubcoreMesh(core_axis_name="c", subcore_axis_name="s")

    @functools.partial(
        pl.kernel,
        out_type=jax.ShapeDtypeStruct((_NW, 16), jnp.float32),
        mesh=mesh,
        scratch_types=[
            pltpu.VMEM((_CH, _DIM), jnp.float32),
            pltpu.VMEM((_CH, _DIM), jnp.float32),
            pltpu.SemaphoreType.DMA,
            pltpu.SemaphoreType.DMA,
        ],
    )
    def body(x_hbm, o_hbm, buf0, buf1, sem0, sem1):
        wid = lax.axis_index("s") * 2 + lax.axis_index("c")
        base = wid * _RPW
        bufs = (buf0, buf1)
        sems = (sem0, sem1)
        prev = pltpu.async_copy(x_hbm.at[pl.ds(base, _CH)], buf0, sem0)
        for j in range(1, _NCH):
            cur = pltpu.async_copy(
                x_hbm.at[pl.ds(base + j * _CH, _CH)], bufs[j % 2], sems[j % 2])
            prev.wait()
            prev = cur
        prev.wait()
        pltpu.sync_copy(bufs[(_NCH - 1) % 2].at[0, pl.ds(0, 16)],
                        o_hbm.at[wid])

    return body(x)


@jax.jit
def kernel(x, expert_bias, W):
    wt = W.T
    bias_col = expert_bias.reshape(_NUM_EXPERTS, 1)
    grid = _NUM_TOKENS // _BLK
    out_k, cnt = pl.pallas_call(
        _router_body,
        grid=(grid,),
        in_specs=[
            pl.BlockSpec((_BLK, _DIM), lambda i: (i, 0)),
            pl.BlockSpec((_DIM, _NUM_EXPERTS), lambda i: (0, 0)),
            pl.BlockSpec((_NUM_EXPERTS, 1), lambda i: (0, 0)),
        ],
        out_specs=[
            pl.BlockSpec((4, _BLK), lambda i: (0, i)),
            pl.BlockSpec((_NUM_EXPERTS, 1), lambda i: (0, 0)),
        ],
        out_shape=[
            jax.ShapeDtypeStruct((4, _NUM_TOKENS), jnp.float32),
            jax.ShapeDtypeStruct((_NUM_EXPERTS, 1), jnp.float32),
        ],
    )(x, wt, bias_col)
    packed = out_k.T
    ts = packed[:, :_TOP_K]
    idx = jax.lax.bitcast_convert_type(packed[:, _TOP_K:], jnp.int32)
    sc = _sc_stream_probe(x)
    return ts, idx, cnt.reshape(_NUM_EXPERTS) + jnp.sum(sc) * 1e-30


# speedup vs baseline: 1.3965x; 1.3965x over previous
import functools

import jax
import jax.numpy as jnp
from jax.experimental import pallas as pl
from jax.experimental.pallas import tpu as pltpu

_NUM_TOKENS = 32768
_DIM = 2048
_NUM_EXPERTS = 8
_TOP_K = 2
_BLK = 1024


def _router_body(x_ref, wt_ref, bias_ref, out_ref, cnt_ref):
    i = pl.program_id(0)
    x = x_ref[...]
    wt = wt_ref[...]
    logits = jax.lax.dot_general(
        x, wt, (((1,), (0,)), ((), ())),
        preferred_element_type=jnp.float32,
    )
    lt = logits.T
    scores = jax.nn.sigmoid(lt)
    biased = scores + bias_ref[...]

    row = jax.lax.broadcasted_iota(jnp.int32, biased.shape, 0)
    m1 = jnp.max(biased, axis=0, keepdims=True)
    i1 = jnp.min(jnp.where(biased == m1, row, _NUM_EXPERTS),
                 axis=0, keepdims=True)
    masked = jnp.where(row == i1, -jnp.inf, biased)
    m2 = jnp.max(masked, axis=0, keepdims=True)
    i2 = jnp.min(jnp.where(masked == m2, row, _NUM_EXPERTS),
                 axis=0, keepdims=True)

    sel1 = row == i1
    sel2 = row == i2
    raw1 = jnp.sum(jnp.where(sel1, scores, 0.0), axis=0, keepdims=True)
    raw2 = jnp.sum(jnp.where(sel2, scores, 0.0), axis=0, keepdims=True)
    denom = raw1 + raw2 + 1e-20
    out_ref[...] = jnp.concatenate(
        [raw1 / denom, raw2 / denom,
         jax.lax.bitcast_convert_type(i1, jnp.float32),
         jax.lax.bitcast_convert_type(i2, jnp.float32)], axis=0)

    counts = jnp.sum(
        jnp.where(sel1, 1.0, 0.0) + jnp.where(sel2, 1.0, 0.0),
        axis=1, keepdims=True)

    @pl.when(i == 0)
    def _init():
        cnt_ref[...] = counts

    @pl.when(i != 0)
    def _accum():
        cnt_ref[...] += counts


@jax.jit
def kernel(x, expert_bias, W):
    wt = W.T
    bias_col = expert_bias.reshape(_NUM_EXPERTS, 1)
    grid = _NUM_TOKENS // _BLK
    out_k, cnt = pl.pallas_call(
        _router_body,
        grid=(grid,),
        in_specs=[
            pl.BlockSpec((_BLK, _DIM), lambda i: (i, 0)),
            pl.BlockSpec((_DIM, _NUM_EXPERTS), lambda i: (0, 0)),
            pl.BlockSpec((_NUM_EXPERTS, 1), lambda i: (0, 0)),
        ],
        out_specs=[
            pl.BlockSpec((4, _BLK), lambda i: (0, i)),
            pl.BlockSpec((_NUM_EXPERTS, 1), lambda i: (0, 0)),
        ],
        out_shape=[
            jax.ShapeDtypeStruct((4, _NUM_TOKENS), jnp.float32),
            jax.ShapeDtypeStruct((_NUM_EXPERTS, 1), jnp.float32),
        ],
    )(x, wt, bias_col)
    packed = out_k.T
    ts = packed[:, :_TOP_K]
    idx = jax.lax.bitcast_convert_type(packed[:, _TOP_K:], jnp.int32)
    return ts, idx, cnt.reshape(_NUM_EXPERTS)


# speedup vs baseline: 1.4019x; 1.0039x over previous
import functools

import jax
import jax.numpy as jnp
from jax.experimental import pallas as pl
from jax.experimental.pallas import tpu as pltpu

_NUM_TOKENS = 32768
_DIM = 2048
_NUM_EXPERTS = 8
_TOP_K = 2
_BLK = 2048


def _router_body(x_ref, wt_ref, bias_ref, out_ref, cnt_ref):
    i = pl.program_id(0)
    x = x_ref[...]
    wt = wt_ref[...]
    logits = jax.lax.dot_general(
        x, wt, (((1,), (0,)), ((), ())),
        preferred_element_type=jnp.float32,
    )
    lt = logits.T
    scores = jax.nn.sigmoid(lt)
    biased = scores + bias_ref[...]

    row = jax.lax.broadcasted_iota(jnp.int32, biased.shape, 0)
    m1 = jnp.max(biased, axis=0, keepdims=True)
    i1 = jnp.min(jnp.where(biased == m1, row, _NUM_EXPERTS),
                 axis=0, keepdims=True)
    masked = jnp.where(row == i1, -jnp.inf, biased)
    m2 = jnp.max(masked, axis=0, keepdims=True)
    i2 = jnp.min(jnp.where(masked == m2, row, _NUM_EXPERTS),
                 axis=0, keepdims=True)

    sel1 = row == i1
    sel2 = row == i2
    raw1 = jnp.sum(jnp.where(sel1, scores, 0.0), axis=0, keepdims=True)
    raw2 = jnp.sum(jnp.where(sel2, scores, 0.0), axis=0, keepdims=True)
    denom = raw1 + raw2 + 1e-20
    out_ref[...] = jnp.concatenate(
        [raw1 / denom, raw2 / denom,
         jax.lax.bitcast_convert_type(i1, jnp.float32),
         jax.lax.bitcast_convert_type(i2, jnp.float32)], axis=0)

    counts = jnp.sum(
        jnp.where(sel1, 1.0, 0.0) + jnp.where(sel2, 1.0, 0.0),
        axis=1, keepdims=True)

    @pl.when(i == 0)
    def _init():
        cnt_ref[...] = counts

    @pl.when(i != 0)
    def _accum():
        cnt_ref[...] += counts


@jax.jit
def kernel(x, expert_bias, W):
    wt = W.T
    bias_col = expert_bias.reshape(_NUM_EXPERTS, 1)
    grid = _NUM_TOKENS // _BLK
    out_k, cnt = pl.pallas_call(
        _router_body,
        grid=(grid,),
        in_specs=[
            pl.BlockSpec((_BLK, _DIM), lambda i: (i, 0)),
            pl.BlockSpec((_DIM, _NUM_EXPERTS), lambda i: (0, 0)),
            pl.BlockSpec((_NUM_EXPERTS, 1), lambda i: (0, 0)),
        ],
        out_specs=[
            pl.BlockSpec((4, _BLK), lambda i: (0, i)),
            pl.BlockSpec((_NUM_EXPERTS, 1), lambda i: (0, 0)),
        ],
        out_shape=[
            jax.ShapeDtypeStruct((4, _NUM_TOKENS), jnp.float32),
            jax.ShapeDtypeStruct((_NUM_EXPERTS, 1), jnp.float32),
        ],
    )(x, wt, bias_col)
    packed = out_k.T
    ts = packed[:, :_TOP_K]
    idx = jax.lax.bitcast_convert_type(packed[:, _TOP_K:], jnp.int32)
    return ts, idx, cnt.reshape(_NUM_EXPERTS)
